# baseline (speedup 1.0000x reference)
import jax
import jax.numpy as jnp
from jax import lax
from jax.experimental import pallas as pl
from jax.experimental.pallas import tpu as pltpu
from jax.experimental.pallas import tpu_sc as plsc

_E = 320000
_N = 10000
_H = 128
_NG = 512
_EPS = 1e-5

_NC = 2
_NS = 16
_NW = _NC * _NS
_EPW = _E // _NW
_B = 80
_NBLK = _EPW // _B


_DEPTH = 8
_LOOP_HI = ((_NBLK + _DEPTH - 1) // _DEPTH) * _DEPTH
_NREG = 1
_RPT = _NG // _NS


def _sc_agg_body(edge_hbm, ei0_hbm, batch_hbm, zeros_hbm, out_hbm,
                 batch_v, ei_v, *rest):
    rows = rest[0:_DEPTH]
    gids = rest[_DEPTH:2 * _DEPTH]
    dsems = rest[2 * _DEPTH:3 * _DEPTH]
    ssems = rest[3 * _DEPTH:4 * _DEPTH]
    tmp_v, idx_v, acc_sh = rest[4 * _DEPTH:]
    c = lax.axis_index("c")
    s = lax.axis_index("s")
    wid = s * _NC + c
    ebase = wid * _EPW

    pltpu.sync_copy(batch_hbm, batch_v)
    pltpu.sync_copy(ei0_hbm.at[pl.ds(ebase, _EPW)], ei_v)

    @pl.when(s < _NREG)
    def _zero():
        pltpu.sync_copy(zeros_hbm, acc_sh.at[pl.ds(s * _NG, _NG)])

    plsc.subcore_barrier()

    reg_off = (s % _NREG) * _NG

    def _start_in(b, par):
        pltpu.async_copy(edge_hbm.at[pl.ds(ebase + b * _B, _B)],
                         rows[par], dsems[par])

    def _wait_in(b, par):
        pltpu.make_async_copy(edge_hbm.at[pl.ds(ebase + b * _B, _B)],
                              rows[par], dsems[par]).wait()

    def _wait_scat(par):
        pltpu.make_async_copy(rows[par], acc_sh.at[gids[par]],
                              ssems[par]).wait()

    _start_in(0, 0)

    @pl.loop(0, _LOOP_HI, step=_DEPTH)
    def _round(b0):
        for par in range(_DEPTH):
            b = b0 + par

            @pl.when(jnp.logical_and(b >= _DEPTH - 1, b + 1 < _NBLK))
            def _pf():
                _wait_scat((par + 1) % _DEPTH)
                _start_in(b + 1, (par + 1) % _DEPTH)

            @pl.when(b + 1 < _DEPTH)
            def _pf0():
                _start_in(b + 1, (par + 1) % _DEPTH)

            @pl.when(b < _NBLK)
            def _do():
                _wait_in(b, par)
                boff = b * _B
                for j in range(_B // 16):
                    idx16 = ei_v[pl.ds(boff + j * 16, 16)]
                    gids[par][pl.ds(j * 16, 16)] = (
                        plsc.load_gather(batch_v, [idx16]) + reg_off)
                pltpu.async_copy(rows[par], acc_sh.at[gids[par]],
                                 ssems[par], add=True)


    for b in range(_NBLK - _DEPTH, _NBLK):
        _wait_scat(b % _DEPTH)

    plsc.subcore_barrier()

    for j in range(_RPT // 16):
        idx_v[pl.ds(j * 16, 16)] = (
            lax.iota(jnp.int32, 16) + (j * 16) + s * _RPT)
    for r in range(1, _NREG):
        pltpu.sync_copy(acc_sh.at[pl.ds(r * _NG + s * _RPT, _RPT)], tmp_v)
        pltpu.sync_copy(tmp_v, acc_sh.at[idx_v], add=True)

    plsc.subcore_barrier()

    @pl.when(s == 0)
    def _flush():
        pltpu.sync_copy(acc_sh.at[pl.ds(0, _NG)], out_hbm.at[c])


def _dense_body(p_ref, w1_ref, w2_ref, w3_ref, gamma_ref, beta_ref,
                mean_ref, var_ref, out_ref):
    g = p_ref[0] + p_ref[1]
    cdims = (((1,), (1,)), ((), ()))
    h = lax.dot_general(g, w1_ref[...], cdims,
                        preferred_element_type=jnp.float32)
    h = lax.dot_general(h, w2_ref[...], cdims,
                        preferred_element_type=jnp.float32)
    h = (h - mean_ref[...]) * lax.rsqrt(var_ref[...] + _EPS)
    h = h * gamma_ref[...] + beta_ref[...]
    h = jax.nn.gelu(h)
    out_ref[...] = lax.dot_general(h, w3_ref[...], cdims,
                                   preferred_element_type=jnp.float32)


def kernel(edge, batch, edge_index, W1, W2, W3, gamma, beta,
           running_mean, running_var):
    zeros = jnp.zeros((_NG, _H), jnp.float32)

    sc_call = pl.kernel(
        _sc_agg_body,
        out_type=jax.ShapeDtypeStruct((_NC, _NG, _H), jnp.float32),
        mesh=plsc.VectorSubcoreMesh(core_axis_name="c", subcore_axis_name="s"),
        scratch_types=(
            [pltpu.VMEM((_N,), jnp.int32),
             pltpu.VMEM((_EPW,), jnp.int32)]
            + [pltpu.VMEM((_B, _H), jnp.float32)] * _DEPTH
            + [pltpu.VMEM((_B,), jnp.int32)] * _DEPTH
            + [pltpu.SemaphoreType.DMA] * (2 * _DEPTH)
            + [pltpu.VMEM((_RPT, _H), jnp.float32),
               pltpu.VMEM((_RPT,), jnp.int32),
               pltpu.VMEM_SHARED((_NREG * _NG, _H), jnp.float32)]
        ),
        compiler_params=pltpu.CompilerParams(needs_layout_passes=False),
    )
    partials = sc_call(edge, edge_index.reshape(-1), batch, zeros)

    out = pl.pallas_call(
        _dense_body,
        out_shape=jax.ShapeDtypeStruct((_NG, _H), jnp.float32),
    )(partials, W1, W2, W3,
      gamma.reshape(1, _H), beta.reshape(1, _H),
      running_mean.reshape(1, _H), running_var.reshape(1, _H))
    return out


# speedup vs baseline: 1.1066x; 1.1066x over previous
import jax
import jax.numpy as jnp
from jax import lax
from jax.experimental import pallas as pl
from jax.experimental.pallas import tpu as pltpu
from jax.experimental.pallas import tpu_sc as plsc

_E = 320000
_N = 10000
_H = 128
_NG = 512
_EPS = 1e-5

_NC = 2
_NS = 16
_NW = _NC * _NS
_EPW = _E // _NW
_B = 80
_NBLK = _EPW // _B


_SUBS = 5
_BIGB = _SUBS * _B
_NBIG = _EPW // _BIGB
_NREG = 1
_RPT = _NG // _NS


def _sc_agg_body(edge_hbm, ei0_hbm, batch_hbm, zeros_hbm, out_hbm,
                 batch_v, ei_v, *rest):
    rows = rest[0:2]
    gids = (rest[2:2 + _SUBS], rest[2 + _SUBS:2 + 2 * _SUBS])
    dsems = rest[2 + 2 * _SUBS:4 + 2 * _SUBS]
    ssems = rest[4 + 2 * _SUBS:6 + 2 * _SUBS]
    tmp_v, idx_v, acc_sh = rest[6 + 2 * _SUBS:]
    c = lax.axis_index("c")
    s = lax.axis_index("s")
    wid = s * _NC + c
    ebase = wid * _EPW

    pltpu.sync_copy(batch_hbm, batch_v)
    pltpu.sync_copy(ei0_hbm.at[pl.ds(ebase, _EPW)], ei_v)

    @pl.when(s < _NREG)
    def _zero():
        pltpu.sync_copy(zeros_hbm, acc_sh.at[pl.ds(s * _NG, _NG)])

    plsc.subcore_barrier()

    reg_off = (s % _NREG) * _NG

    def _start_in(bb, par):
        pltpu.async_copy(edge_hbm.at[pl.ds(ebase + bb * _BIGB, _BIGB)],
                         rows[par], dsems[par])

    def _wait_in(bb, par):
        pltpu.make_async_copy(edge_hbm.at[pl.ds(ebase + bb * _BIGB, _BIGB)],
                              rows[par], dsems[par]).wait()

    def _wait_scat(par, k):
        pltpu.make_async_copy(rows[par].at[pl.ds(k * _B, _B)],
                              acc_sh.at[gids[par][k]], ssems[par]).wait()

    _start_in(0, 0)

    @pl.loop(0, _NBIG + 1, step=2)
    def _round(bb0):
        for par in range(2):
            bb = bb0 + par
            other = 1 - par

            @pl.when(bb + 1 < _NBIG)
            def _pf():
                @pl.when(bb >= 1)
                def _dr():
                    for k in range(_SUBS):
                        _wait_scat(other, k)
                _start_in(bb + 1, other)

            @pl.when(bb < _NBIG)
            def _do():
                _wait_in(bb, par)
                for k in range(_SUBS):
                    for j in range(_B // 16):
                        idx16 = ei_v[pl.ds(bb * _BIGB + k * _B + j * 16, 16)]
                        gids[par][k][pl.ds(j * 16, 16)] = (
                            plsc.load_gather(batch_v, [idx16]) + reg_off)
                    pltpu.async_copy(rows[par].at[pl.ds(k * _B, _B)],
                                     acc_sh.at[gids[par][k]],
                                     ssems[par], add=True)

    for bb in (_NBIG - 2, _NBIG - 1):
        for k in range(_SUBS):
            _wait_scat(bb % 2, k)

    plsc.subcore_barrier()

    for j in range(_RPT // 16):
        idx_v[pl.ds(j * 16, 16)] = (
            lax.iota(jnp.int32, 16) + (j * 16) + s * _RPT)
    for r in range(1, _NREG):
        pltpu.sync_copy(acc_sh.at[pl.ds(r * _NG + s * _RPT, _RPT)], tmp_v)
        pltpu.sync_copy(tmp_v, acc_sh.at[idx_v], add=True)

    plsc.subcore_barrier()

    @pl.when(s == 0)
    def _flush():
        pltpu.sync_copy(acc_sh.at[pl.ds(0, _NG)], out_hbm.at[c])


def _dense_body(p_ref, w1_ref, w2_ref, w3_ref, gamma_ref, beta_ref,
                mean_ref, var_ref, out_ref):
    g = p_ref[0] + p_ref[1]
    cdims = (((1,), (1,)), ((), ()))
    h = lax.dot_general(g, w1_ref[...], cdims,
                        preferred_element_type=jnp.float32)
    h = lax.dot_general(h, w2_ref[...], cdims,
                        preferred_element_type=jnp.float32)
    h = (h - mean_ref[...]) * lax.rsqrt(var_ref[...] + _EPS)
    h = h * gamma_ref[...] + beta_ref[...]
    h = jax.nn.gelu(h)
    out_ref[...] = lax.dot_general(h, w3_ref[...], cdims,
                                   preferred_element_type=jnp.float32)


def kernel(edge, batch, edge_index, W1, W2, W3, gamma, beta,
           running_mean, running_var):
    zeros = jnp.zeros((_NG, _H), jnp.float32)

    sc_call = pl.kernel(
        _sc_agg_body,
        out_type=jax.ShapeDtypeStruct((_NC, _NG, _H), jnp.float32),
        mesh=plsc.VectorSubcoreMesh(core_axis_name="c", subcore_axis_name="s"),
        scratch_types=(
            [pltpu.VMEM((_N,), jnp.int32),
             pltpu.VMEM((_EPW,), jnp.int32)]
            + [pltpu.VMEM((_BIGB, _H), jnp.float32)] * 2
            + [pltpu.VMEM((_B,), jnp.int32)] * (2 * _SUBS)
            + [pltpu.SemaphoreType.DMA] * 4
            + [pltpu.VMEM((_RPT, _H), jnp.float32),
               pltpu.VMEM((_RPT,), jnp.int32),
               pltpu.VMEM_SHARED((_NREG * _NG, _H), jnp.float32)]
        ),
        compiler_params=pltpu.CompilerParams(needs_layout_passes=False),
    )
    partials = sc_call(edge, edge_index.reshape(-1), batch, zeros)

    out = pl.pallas_call(
        _dense_body,
        out_shape=jax.ShapeDtypeStruct((_NG, _H), jnp.float32),
    )(partials, W1, W2, W3,
      gamma.reshape(1, _H), beta.reshape(1, _H),
      running_mean.reshape(1, _H), running_var.reshape(1, _H))
    return out


# speedup vs baseline: 1.1136x; 1.0064x over previous
import jax
import jax.numpy as jnp
from jax import lax
from jax.experimental import pallas as pl
from jax.experimental.pallas import tpu as pltpu
from jax.experimental.pallas import tpu_sc as plsc

_E = 320000
_N = 10000
_H = 128
_NG = 512
_EPS = 1e-5

_NC = 2
_NS = 16
_NW = _NC * _NS
_EPW = _E // _NW
_B = 80
_NBLK = _EPW // _B


_SUBS = 5
_BIGB = _SUBS * _B
_NBIG = _EPW // _BIGB
_NREG = 1
_RPT = _NG // _NS


def _sc_agg_body(edge_hbm, ei0_hbm, batch_hbm, zeros_hbm, out_hbm,
                 batch_v, ei_v, *rest):
    rows = rest[0:2]
    gids = (rest[2:2 + _SUBS], rest[2 + _SUBS:2 + 2 * _SUBS])
    dsems = rest[2 + 2 * _SUBS:4 + 2 * _SUBS]
    ssems = rest[4 + 2 * _SUBS:6 + 2 * _SUBS]
    tmp_v, idx_v, acc_sh = rest[6 + 2 * _SUBS:]
    c = lax.axis_index("c")
    s = lax.axis_index("s")
    wid = s * _NC + c
    ebase = wid * _EPW

    pltpu.sync_copy(batch_hbm, batch_v)
    pltpu.sync_copy(ei0_hbm.at[pl.ds(ebase, _EPW)], ei_v)

    @pl.when(s < _NREG)
    def _zero():
        pltpu.sync_copy(zeros_hbm, acc_sh.at[pl.ds(s * _NG, _NG)])

    plsc.subcore_barrier()

    reg_off = (s % _NREG) * _NG

    def _start_in(bb, par):
        pltpu.async_copy(edge_hbm.at[pl.ds(ebase + bb * _BIGB, _BIGB)],
                         rows[par], dsems[par])

    def _wait_in(bb, par):
        pltpu.make_async_copy(edge_hbm.at[pl.ds(ebase + bb * _BIGB, _BIGB)],
                              rows[par], dsems[par]).wait()

    def _wait_scat(par, k):
        pltpu.make_async_copy(rows[par].at[pl.ds(k * _B, _B)],
                              acc_sh.at[gids[par][k]], ssems[par]).wait()

    _start_in(0, 0)

    @pl.loop(0, _NBIG + 1, step=2)
    def _round(bb0):
        for par in range(2):
            bb = bb0 + par
            other = 1 - par

            @pl.when(bb + 1 < _NBIG)
            def _pf():
                @pl.when(bb >= 1)
                def _dr():
                    for k in range(_SUBS):
                        _wait_scat(other, k)
                _start_in(bb + 1, other)

            @pl.when(bb < _NBIG)
            def _do():
                for k in range(_SUBS):
                    for j in range(_B // 16):
                        idx16 = ei_v[pl.ds(bb * _BIGB + k * _B + j * 16, 16)]
                        gids[par][k][pl.ds(j * 16, 16)] = (
                            plsc.load_gather(batch_v, [idx16]) + reg_off)
                _wait_in(bb, par)
                for k in range(_SUBS):
                    pltpu.async_copy(rows[par].at[pl.ds(k * _B, _B)],
                                     acc_sh.at[gids[par][k]],
                                     ssems[par], add=True)

    for bb in (_NBIG - 2, _NBIG - 1):
        for k in range(_SUBS):
            _wait_scat(bb % 2, k)

    plsc.subcore_barrier()

    for j in range(_RPT // 16):
        idx_v[pl.ds(j * 16, 16)] = (
            lax.iota(jnp.int32, 16) + (j * 16) + s * _RPT)
    for r in range(1, _NREG):
        pltpu.sync_copy(acc_sh.at[pl.ds(r * _NG + s * _RPT, _RPT)], tmp_v)
        pltpu.sync_copy(tmp_v, acc_sh.at[idx_v], add=True)

    plsc.subcore_barrier()

    @pl.when(s == 0)
    def _flush():
        pltpu.sync_copy(acc_sh.at[pl.ds(0, _NG)], out_hbm.at[c])


def _dense_body(p_ref, w1_ref, w2_ref, w3_ref, gamma_ref, beta_ref,
                mean_ref, var_ref, out_ref):
    g = p_ref[0] + p_ref[1]
    cdims = (((1,), (1,)), ((), ()))
    h = lax.dot_general(g, w1_ref[...], cdims,
                        preferred_element_type=jnp.float32)
    h = lax.dot_general(h, w2_ref[...], cdims,
                        preferred_element_type=jnp.float32)
    h = (h - mean_ref[...]) * lax.rsqrt(var_ref[...] + _EPS)
    h = h * gamma_ref[...] + beta_ref[...]
    h = jax.nn.gelu(h)
    out_ref[...] = lax.dot_general(h, w3_ref[...], cdims,
                                   preferred_element_type=jnp.float32)


def kernel(edge, batch, edge_index, W1, W2, W3, gamma, beta,
           running_mean, running_var):
    zeros = jnp.zeros((_NG, _H), jnp.float32)

    sc_call = pl.kernel(
        _sc_agg_body,
        out_type=jax.ShapeDtypeStruct((_NC, _NG, _H), jnp.float32),
        mesh=plsc.VectorSubcoreMesh(core_axis_name="c", subcore_axis_name="s"),
        scratch_types=(
            [pltpu.VMEM((_N,), jnp.int32),
             pltpu.VMEM((_EPW,), jnp.int32)]
            + [pltpu.VMEM((_BIGB, _H), jnp.float32)] * 2
            + [pltpu.VMEM((_B,), jnp.int32)] * (2 * _SUBS)
            + [pltpu.SemaphoreType.DMA] * 4
            + [pltpu.VMEM((_RPT, _H), jnp.float32),
               pltpu.VMEM((_RPT,), jnp.int32),
               pltpu.VMEM_SHARED((_NREG * _NG, _H), jnp.float32)]
        ),
        compiler_params=pltpu.CompilerParams(needs_layout_passes=False),
    )
    partials = sc_call(edge, edge_index.reshape(-1), batch, zeros)

    out = pl.pallas_call(
        _dense_body,
        out_shape=jax.ShapeDtypeStruct((_NG, _H), jnp.float32),
    )(partials, W1, W2, W3,
      gamma.reshape(1, _H), beta.reshape(1, _H),
      running_mean.reshape(1, _H), running_var.reshape(1, _H))
    return out
